# speedup vs baseline: 3.6057x; 3.6057x over previous
import functools

import jax
import jax.numpy as jnp
from jax import lax
from jax.experimental import pallas as pl
from jax.experimental.pallas import tpu as pltpu
from jax.experimental.pallas import tpu_sc as plsc

_NC = 2
_NS = 16
_NW = _NC * _NS


def _gather_kernel(per_w, C, n_chunks, x_hbm, idx_hbm, out_hbm,
                   idx_v, rows_v, sem):
    wid = lax.axis_index("s") * _NC + lax.axis_index("c")
    base = wid * per_w
    pltpu.sync_copy(idx_hbm.at[pl.ds(base, per_w)], idx_v)

    def body(g, _):
        idx_chunk = idx_v.at[pl.ds(g * C, C)]
        pltpu.async_copy(x_hbm.at[idx_chunk], rows_v, sem).wait()
        pltpu.sync_copy(rows_v, out_hbm.at[pl.ds(base + g * C, C)])
        return ()

    lax.fori_loop(0, n_chunks, body, (), unroll=False)


def kernel(x, seed_inverse_ids):
    T = seed_inverse_ids.shape[0]
    D = x.shape[1]
    per_w = T // _NW
    C = 80
    n_chunks = per_w // C
    assert per_w * _NW == T and n_chunks * C == per_w

    run = pl.kernel(
        functools.partial(_gather_kernel, per_w, C, n_chunks),
        out_type=jax.ShapeDtypeStruct((T, D), jnp.float32),
        mesh=plsc.VectorSubcoreMesh(core_axis_name="c", subcore_axis_name="s"),
        scratch_types=[
            pltpu.VMEM((per_w,), jnp.int32),
            pltpu.VMEM((C, D), jnp.float32),
            pltpu.SemaphoreType.DMA,
        ],
    )
    return run(x, seed_inverse_ids)


# speedup vs baseline: 5.5157x; 1.5297x over previous
import functools

import jax
import jax.numpy as jnp
from jax import lax
from jax.experimental import pallas as pl
from jax.experimental.pallas import tpu as pltpu
from jax.experimental.pallas import tpu_sc as plsc

_NC = 2
_NS = 16
_NW = _NC * _NS


def _gather_kernel(per_w, C, n_chunks, x_hbm, idx_hbm, out_hbm,
                   idx_v, rows_v, sem0, sem1):
    wid = lax.axis_index("s") * _NC + lax.axis_index("c")
    base = wid * per_w
    pltpu.sync_copy(idx_hbm.at[pl.ds(base, per_w)], idx_v)

    def fire(g, b, sem):
        pltpu.async_copy(x_hbm.at[idx_v.at[pl.ds(g * C, C)]],
                         rows_v.at[b], sem)

    def wait_and_store(g, b, sem):
        pltpu.make_async_copy(x_hbm.at[idx_v.at[pl.ds(g * C, C)]],
                              rows_v.at[b], sem).wait()
        pltpu.sync_copy(rows_v.at[b], out_hbm.at[pl.ds(base + g * C, C)])

    fire(0, 0, sem0)

    def body(g2, _):
        g = g2 * 2
        fire(g + 1, 1, sem1)
        wait_and_store(g, 0, sem0)
        fire(g + 2, 0, sem0)
        wait_and_store(g + 1, 1, sem1)
        return ()

    lax.fori_loop(0, (n_chunks - 1) // 2, body, (), unroll=False)
    wait_and_store(n_chunks - 1, 0, sem0)


def kernel(x, seed_inverse_ids):
    T = seed_inverse_ids.shape[0]
    D = x.shape[1]
    per_w = T // _NW
    C = 80
    n_chunks = per_w // C
    assert per_w * _NW == T and n_chunks * C == per_w and n_chunks % 2 == 1

    run = pl.kernel(
        functools.partial(_gather_kernel, per_w, C, n_chunks),
        out_type=jax.ShapeDtypeStruct((T, D), jnp.float32),
        mesh=plsc.VectorSubcoreMesh(core_axis_name="c", subcore_axis_name="s"),
        scratch_types=[
            pltpu.VMEM((per_w,), jnp.int32),
            pltpu.VMEM((2, C, D), jnp.float32),
            pltpu.SemaphoreType.DMA,
            pltpu.SemaphoreType.DMA,
        ],
    )
    return run(x, seed_inverse_ids)


# speedup vs baseline: 6.0812x; 1.1025x over previous
import functools

import jax
import jax.numpy as jnp
from jax import lax
from jax.experimental import pallas as pl
from jax.experimental.pallas import tpu as pltpu
from jax.experimental.pallas import tpu_sc as plsc

_NC = 2
_NS = 16
_NW = _NC * _NS
_NBUF = 4


def _gather_kernel(per_w, C, n_chunks, x_hbm, idx_hbm, out_hbm,
                   idx_v, rows_v, sems_in, sems_out):
    wid = lax.axis_index("s") * _NC + lax.axis_index("c")
    base = wid * per_w
    pltpu.sync_copy(idx_hbm.at[pl.ds(base, per_w)], idx_v)

    def g_copy(g, b, start):
        cp = pltpu.make_async_copy(x_hbm.at[idx_v.at[pl.ds(g * C, C)]],
                                   rows_v.at[b], sems_in[b])
        cp.start() if start else cp.wait()

    def s_copy(g, b, start):
        cp = pltpu.make_async_copy(rows_v.at[b],
                                   out_hbm.at[pl.ds(base + g * C, C)],
                                   sems_out[b])
        cp.start() if start else cp.wait()

    def step(g, slot, do_wait_s, do_fire_g):
        if do_wait_s:
            s_copy(g - 2, (slot + 2) % _NBUF, False)
        if do_fire_g:
            g_copy(g + 2, (slot + 2) % _NBUF, True)
        g_copy(g, slot, False)
        s_copy(g, slot, True)

    g_copy(0, 0, True)
    g_copy(1, 1, True)
    step(0, 0, False, True)
    step(1, 1, False, True)

    n_main = (n_chunks - 5) // _NBUF

    def body(g4, _):
        g = 2 + g4 * _NBUF
        for j in range(_NBUF):
            step(g + j, (2 + j) % _NBUF, True, True)
        return ()

    lax.fori_loop(0, n_main, body, (), unroll=False)

    for g in range(2 + n_main * _NBUF, n_chunks):
        step(g, g % _NBUF, True, g + 2 < n_chunks)
    for g in range(n_chunks - 2, n_chunks):
        s_copy(g, g % _NBUF, False)


def kernel(x, seed_inverse_ids):
    T = seed_inverse_ids.shape[0]
    D = x.shape[1]
    per_w = T // _NW
    C = 80
    n_chunks = per_w // C
    assert per_w * _NW == T and n_chunks * C == per_w and n_chunks > 8

    run = pl.kernel(
        functools.partial(_gather_kernel, per_w, C, n_chunks),
        out_type=jax.ShapeDtypeStruct((T, D), jnp.float32),
        mesh=plsc.VectorSubcoreMesh(core_axis_name="c", subcore_axis_name="s"),
        scratch_types=[
            pltpu.VMEM((per_w,), jnp.int32),
            pltpu.VMEM((_NBUF, C, D), jnp.float32),
            [pltpu.SemaphoreType.DMA] * _NBUF,
            [pltpu.SemaphoreType.DMA] * _NBUF,
        ],
    )
    return run(x, seed_inverse_ids)


# speedup vs baseline: 6.0938x; 1.0021x over previous
import functools

import jax
import jax.numpy as jnp
from jax import lax
from jax.experimental import pallas as pl
from jax.experimental.pallas import tpu as pltpu
from jax.experimental.pallas import tpu_sc as plsc

_NC = 2
_NS = 16
_NW = _NC * _NS
_NBUF = 4


def _gather_kernel(per_w, C, n_full, tail, x_hbm, idx_hbm, out_hbm,
                   idx_v, rows_v, sems_in, sems_out):
    n_chunks = n_full + (1 if tail else 0)
    wid = lax.axis_index("s") * _NC + lax.axis_index("c")
    base = wid * per_w
    pltpu.sync_copy(idx_hbm.at[pl.ds(base, per_w)], idx_v)

    def g_copy(g, b, start, sz=C):
        cp = pltpu.make_async_copy(x_hbm.at[idx_v.at[pl.ds(g * C, sz)]],
                                   rows_v.at[b].at[pl.ds(0, sz)], sems_in[b])
        cp.start() if start else cp.wait()

    def s_copy(g, b, start, sz=C):
        cp = pltpu.make_async_copy(rows_v.at[b].at[pl.ds(0, sz)],
                                   out_hbm.at[pl.ds(base + g * C, sz)],
                                   sems_out[b])
        cp.start() if start else cp.wait()

    def size_of(g):
        return tail if (tail and g == n_chunks - 1) else C

    def step(g, slot, do_wait_s, do_fire_g, sz=C, sz_prev=C, sz_next=C):
        if do_wait_s:
            s_copy(g - 2, (slot + 2) % _NBUF, False, sz_prev)
        if do_fire_g:
            g_copy(g + 2, (slot + 2) % _NBUF, True, sz_next)
        g_copy(g, slot, False, sz)
        s_copy(g, slot, True, sz)

    g_copy(0, 0, True, size_of(0))
    g_copy(1, 1, True, size_of(1))
    step(0, 0, False, True, size_of(0), C, size_of(2))
    step(1, 1, False, True, size_of(1), C, size_of(3))

    n_main = max(0, (n_chunks - 5) // _NBUF)
    while 2 + _NBUF * n_main + 2 > n_full:
        n_main -= 1

    def body(g4, _):
        g = 2 + g4 * _NBUF
        for j in range(_NBUF):
            step(g + j, (2 + j) % _NBUF, True, True)
        return ()

    lax.fori_loop(0, n_main, body, (), unroll=False)

    for g in range(2 + n_main * _NBUF, n_chunks):
        step(g, g % _NBUF, True, g + 2 < n_chunks,
             size_of(g), C, size_of(g + 2))
    for g in range(n_chunks - 2, n_chunks):
        s_copy(g, g % _NBUF, False, size_of(g))


def kernel(x, seed_inverse_ids):
    T = seed_inverse_ids.shape[0]
    D = x.shape[1]
    per_w = T // _NW
    C = 128
    n_full = per_w // C
    tail = per_w - n_full * C
    assert per_w * _NW == T and tail % 8 == 0 and n_full > 8

    run = pl.kernel(
        functools.partial(_gather_kernel, per_w, C, n_full, tail),
        out_type=jax.ShapeDtypeStruct((T, D), jnp.float32),
        mesh=plsc.VectorSubcoreMesh(core_axis_name="c", subcore_axis_name="s"),
        scratch_types=[
            pltpu.VMEM((per_w,), jnp.int32),
            pltpu.VMEM((_NBUF, C, D), jnp.float32),
            [pltpu.SemaphoreType.DMA] * _NBUF,
            [pltpu.SemaphoreType.DMA] * _NBUF,
        ],
    )
    return run(x, seed_inverse_ids)


# speedup vs baseline: 9.6491x; 1.5834x over previous
import functools

import jax
import jax.numpy as jnp
from jax import lax
from jax.experimental import pallas as pl
from jax.experimental.pallas import tpu as pltpu
from jax.experimental.pallas import tpu_sc as plsc

_NC = 2
_NS = 16
_NW = _NC * _NS
_NBUF = 4


def _gather_kernel(per_w, C, n_full, tail, n_rows, x_hbm, idx_hbm, out_hbm,
                   table_sh, idx_v, rows_v, sems_in, sems_out, sem_stage):
    n_chunks = n_full + (1 if tail else 0)
    cid = lax.axis_index("c")
    sid = lax.axis_index("s")
    wid = sid * _NC + cid
    base = wid * per_w

    stage_rows = (n_rows // _NS) // 8 * 8
    rem_rows = n_rows - stage_rows * _NS
    pltpu.async_copy(x_hbm.at[pl.ds(sid * stage_rows, stage_rows)],
                     table_sh.at[pl.ds(sid * stage_rows, stage_rows)],
                     sem_stage)
    if rem_rows:
        @pl.when(sid == _NS - 1)
        def _():
            pltpu.async_copy(
                x_hbm.at[pl.ds(stage_rows * _NS, rem_rows)],
                table_sh.at[pl.ds(stage_rows * _NS, rem_rows)], sem_stage)
    pltpu.sync_copy(idx_hbm.at[pl.ds(base, per_w)], idx_v)
    pltpu.make_async_copy(x_hbm.at[pl.ds(sid * stage_rows, stage_rows)],
                          table_sh.at[pl.ds(sid * stage_rows, stage_rows)],
                          sem_stage).wait()
    if rem_rows:
        @pl.when(sid == _NS - 1)
        def _():
            pltpu.make_async_copy(
                x_hbm.at[pl.ds(stage_rows * _NS, rem_rows)],
                table_sh.at[pl.ds(stage_rows * _NS, rem_rows)],
                sem_stage).wait()
    plsc.subcore_barrier()

    def g_copy(g, b, start, sz=C):
        cp = pltpu.make_async_copy(table_sh.at[idx_v.at[pl.ds(g * C, sz)]],
                                   rows_v.at[b].at[pl.ds(0, sz)], sems_in[b])
        cp.start() if start else cp.wait()

    def s_copy(g, b, start, sz=C):
        cp = pltpu.make_async_copy(rows_v.at[b].at[pl.ds(0, sz)],
                                   out_hbm.at[pl.ds(base + g * C, sz)],
                                   sems_out[b])
        cp.start() if start else cp.wait()

    def size_of(g):
        return tail if (tail and g == n_chunks - 1) else C

    def step(g, slot, do_wait_s, do_fire_g, sz=C, sz_prev=C, sz_next=C):
        if do_wait_s:
            s_copy(g - 2, (slot + 2) % _NBUF, False, sz_prev)
        if do_fire_g:
            g_copy(g + 2, (slot + 2) % _NBUF, True, sz_next)
        g_copy(g, slot, False, sz)
        s_copy(g, slot, True, sz)

    g_copy(0, 0, True, size_of(0))
    g_copy(1, 1, True, size_of(1))
    step(0, 0, False, True, size_of(0), C, size_of(2))
    step(1, 1, False, True, size_of(1), C, size_of(3))

    n_main = max(0, (n_chunks - 5) // _NBUF)
    while 2 + _NBUF * n_main + 2 > n_full:
        n_main -= 1

    def body(g4, _):
        g = 2 + g4 * _NBUF
        for j in range(_NBUF):
            step(g + j, (2 + j) % _NBUF, True, True)
        return ()

    lax.fori_loop(0, n_main, body, (), unroll=False)

    for g in range(2 + n_main * _NBUF, n_chunks):
        step(g, g % _NBUF, True, g + 2 < n_chunks,
             size_of(g), C, size_of(g + 2))
    for g in range(n_chunks - 2, n_chunks):
        s_copy(g, g % _NBUF, False, size_of(g))


def kernel(x, seed_inverse_ids):
    T = seed_inverse_ids.shape[0]
    D = x.shape[1]
    per_w = T // _NW
    C = 64
    n_rows = x.shape[0]
    n_full = per_w // C
    tail = per_w - n_full * C
    assert per_w * _NW == T and tail % 8 == 0 and n_full > 8
    assert n_rows % 8 == 0

    run = pl.kernel(
        functools.partial(_gather_kernel, per_w, C, n_full, tail, n_rows),
        out_type=jax.ShapeDtypeStruct((T, D), jnp.float32),
        mesh=plsc.VectorSubcoreMesh(core_axis_name="c", subcore_axis_name="s"),
        scratch_types=[
            pltpu.VMEM_SHARED((n_rows, D), jnp.float32),
            pltpu.VMEM((per_w,), jnp.int32),
            pltpu.VMEM((_NBUF, C, D), jnp.float32),
            [pltpu.SemaphoreType.DMA] * _NBUF,
            [pltpu.SemaphoreType.DMA] * _NBUF,
            pltpu.SemaphoreType.DMA,
        ],
    )
    return run(x, seed_inverse_ids)
